# Initial kernel scaffold; baseline (speedup 1.0000x reference)
import jax
import jax.numpy as jnp
from jax.experimental import pallas as pl


def kernel(node_ids, edge_index, etype, entity, W1, W2):
    raise NotImplementedError("write your pallas kernel here")


# speedup vs baseline: 12.5684x; 12.5684x over previous
import functools

import jax
import jax.numpy as jnp
from jax import lax
from jax.experimental import pallas as pl
from jax.experimental.pallas import tpu as pltpu
from jax.experimental.pallas import tpu_sc as plsc

N = 10000
E = 320000
D = 128
R = 8
T = 64
NTILES = 32
NPAD = 10240
CHUNK = 128
N_CHUNKS = 79
EPT = N_CHUNKS * CHUNK
E_PAD = NTILES * EPT
ROWS_PER_SUB = NPAD // 16
LANES = 16

_MESH = plsc.VectorSubcoreMesh(core_axis_name="c", subcore_axis_name="s")


def _sc_edge_pass(msg_width, use_nid_table):
    stride = N if use_nid_table else NPAD

    def body(*refs):
        if use_nid_table:
            (nid_hbm, src_hbm, dst_hbm, ety_hbm, tab_hbm, zrow_hbm, zcnt_hbm,
             acc_hbm, cnt_hbm,
             srcbuf, etybuf, dstbuf, rowbuf, nidbuf, msgbuf, onesbuf,
             acc_sh, cnt_sh, sem) = refs
        else:
            (src_hbm, dst_hbm, ety_hbm, tab_hbm, zrow_hbm,
             acc_hbm,
             srcbuf, etybuf, dstbuf, rowbuf, msgbuf,
             acc_sh, sem) = refs
        c = lax.axis_index("c")
        s = lax.axis_index("s")
        gwid = c * 16 + s

        pltpu.sync_copy(zrow_hbm, acc_sh.at[pl.ds(s * ROWS_PER_SUB, ROWS_PER_SUB)])
        if use_nid_table:
            pltpu.sync_copy(zcnt_hbm, cnt_sh.at[pl.ds(s * ROWS_PER_SUB, ROWS_PER_SUB)])
            for i in range(CHUNK // LANES):
                onesbuf[pl.ds(i * LANES, LANES)] = jnp.ones((LANES,), jnp.float32)
        plsc.subcore_barrier()

        base0 = gwid * EPT

        def chunk_body(j, carry):
            base = base0 + j * CHUNK
            pltpu.sync_copy(src_hbm.at[pl.ds(base, CHUNK)], srcbuf.at[0])
            pltpu.sync_copy(ety_hbm.at[pl.ds(base, CHUNK)], etybuf.at[0])
            pltpu.sync_copy(dst_hbm.at[pl.ds(base, CHUNK)], dstbuf.at[0])
            if use_nid_table:
                pltpu.async_copy(nid_hbm.at[srcbuf.at[0]], nidbuf.at[0], sem).wait()
            for i in range(CHUNK // LANES):
                sl = pl.ds(i * LANES, LANES)
                s16 = nidbuf[0, sl] if use_nid_table else srcbuf[0, sl]
                e16 = etybuf[0, sl]
                rowbuf[0, sl] = e16 * stride + s16
            pltpu.async_copy(tab_hbm.at[rowbuf.at[0]], msgbuf, sem).wait()
            pltpu.sync_copy(msgbuf, acc_sh.at[dstbuf.at[0]], add=True)
            if use_nid_table:
                pltpu.sync_copy(onesbuf, cnt_sh.at[dstbuf.at[0]], add=True)
            return carry

        lax.fori_loop(0, N_CHUNKS, chunk_body, 0)
        plsc.subcore_barrier()

        rows = pl.ds(s * ROWS_PER_SUB, ROWS_PER_SUB)
        pltpu.sync_copy(acc_sh.at[rows], acc_hbm.at[c, rows])
        if use_nid_table:
            pltpu.sync_copy(cnt_sh.at[rows], cnt_hbm.at[c, rows])

    if use_nid_table:
        out_type = (jax.ShapeDtypeStruct((2, NPAD, msg_width), jnp.float32),
                    jax.ShapeDtypeStruct((2, NPAD), jnp.float32))
        scratch = [
            pltpu.VMEM((1, CHUNK), jnp.int32),
            pltpu.VMEM((1, CHUNK), jnp.int32),
            pltpu.VMEM((1, CHUNK), jnp.int32),
            pltpu.VMEM((1, CHUNK), jnp.int32),
            pltpu.VMEM((1, CHUNK), jnp.int32),
            pltpu.VMEM((CHUNK, msg_width), jnp.float32),
            pltpu.VMEM((CHUNK,), jnp.float32),
            pltpu.VMEM_SHARED((NPAD, msg_width), jnp.float32),
            pltpu.VMEM_SHARED((NPAD,), jnp.float32),
            pltpu.SemaphoreType.DMA,
        ]
    else:
        out_type = jax.ShapeDtypeStruct((2, NPAD, msg_width), jnp.float32)
        scratch = [
            pltpu.VMEM((1, CHUNK), jnp.int32),
            pltpu.VMEM((1, CHUNK), jnp.int32),
            pltpu.VMEM((1, CHUNK), jnp.int32),
            pltpu.VMEM((1, CHUNK), jnp.int32),
            pltpu.VMEM((CHUNK, msg_width), jnp.float32),
            pltpu.VMEM_SHARED((NPAD, msg_width), jnp.float32),
            pltpu.SemaphoreType.DMA,
        ]
    return pl.kernel(body, mesh=_MESH, out_type=out_type, scratch_types=scratch)


def _dot(a, b):
    return lax.dot_general(a, b, (((1,), (0,)), ((), ())),
                           preferred_element_type=jnp.float32,
                           precision=lax.Precision.HIGHEST)


def _tc1_body(x_ref, w_ref, o_ref):
    x = x_ref[...]
    for r in range(R):
        o_ref[r] = _dot(x, w_ref[r])


def _tc2_body(a_ref, c_ref, w_ref, o_ref):
    cnt = jnp.maximum(c_ref[0] + c_ref[1], 1.0)
    h = jnp.maximum((a_ref[0] + a_ref[1]) / cnt[:, None], 0.0)
    for r in range(R):
        o_ref[r] = _dot(h, w_ref[r])


def _tc3_body(a_ref, c_ref, o_ref):
    cnt = jnp.maximum(c_ref[0] + c_ref[1], 1.0)
    o_ref[...] = ((a_ref[0] + a_ref[1]) / cnt[:, None])[:, :T]


def _tc1(entity, W1):
    bn = 2000
    return pl.pallas_call(
        _tc1_body,
        grid=(N // bn,),
        in_specs=[
            pl.BlockSpec((bn, D), lambda i: (i, 0)),
            pl.BlockSpec((R, D, D), lambda i: (0, 0, 0)),
        ],
        out_specs=pl.BlockSpec((R, bn, D), lambda i: (0, i, 0)),
        out_shape=jax.ShapeDtypeStruct((R, N, D), jnp.float32),
    )(entity, W1)


def _tc2(acc1, cnt, W2):
    bn = 2048
    return pl.pallas_call(
        _tc2_body,
        grid=(NPAD // bn,),
        in_specs=[
            pl.BlockSpec((2, bn, D), lambda i: (0, i, 0)),
            pl.BlockSpec((2, bn), lambda i: (0, i)),
            pl.BlockSpec((R, D, D), lambda i: (0, 0, 0)),
        ],
        out_specs=pl.BlockSpec((R, bn, D), lambda i: (0, i, 0)),
        out_shape=jax.ShapeDtypeStruct((R, NPAD, D), jnp.float32),
    )(acc1, cnt, W2)


def _tc3(acc2, cnt):
    bn = 2048
    return pl.pallas_call(
        _tc3_body,
        grid=(NPAD // bn,),
        in_specs=[
            pl.BlockSpec((2, bn, D), lambda i: (0, i, 0)),
            pl.BlockSpec((2, bn), lambda i: (0, i)),
        ],
        out_specs=pl.BlockSpec((bn, T), lambda i: (i, 0)),
        out_shape=jax.ShapeDtypeStruct((NPAD, T), jnp.float32),
    )(acc2, cnt)


_sc1 = _sc_edge_pass(D, use_nid_table=True)
_sc2 = _sc_edge_pass(D, use_nid_table=False)


def kernel(node_ids, edge_index, etype, entity, W1, W2):
    node_ids = node_ids.astype(jnp.int32)
    src = edge_index[0].astype(jnp.int32)
    dst = edge_index[1].astype(jnp.int32)
    etype = etype.astype(jnp.int32)
    pad = E_PAD - E
    srcp = jnp.concatenate([src, jnp.zeros((pad,), jnp.int32)])
    dstp = jnp.concatenate([dst, jnp.full((pad,), NPAD - 1, jnp.int32)])
    etyp = jnp.concatenate([etype, jnp.zeros((pad,), jnp.int32)])
    zrow = jnp.zeros((ROWS_PER_SUB, D), jnp.float32)
    zcnt = jnp.zeros((ROWS_PER_SUB,), jnp.float32)
    W2p = jnp.pad(W2, ((0, 0), (0, 0), (0, D - T)))

    ew1 = _tc1(entity, W1).reshape(R * N, D)
    acc1, cnt = _sc1(node_ids, srcp, dstp, etyp, ew1, zrow, zcnt)
    ew2 = _tc2(acc1, cnt, W2p).reshape(R * NPAD, D)
    acc2 = _sc2(srcp, dstp, etyp, ew2, zrow)
    y = _tc3(acc2, cnt)[:N]
    return (y[None], jnp.ones((1, N), jnp.float32))
